# baseline (speedup 1.0000x reference)
import functools

import jax
import jax.numpy as jnp
from jax.experimental import pallas as pl
from jax.experimental.pallas import tpu as pltpu

N_EXP = 8
TOPK = 2


def _grouped_matmul_kernel(be_ref, xs_ref, w_ref, b_ref, wpad_ref, out_ref):
    y = jnp.dot(xs_ref[...], w_ref[0], preferred_element_type=jnp.float32)
    y = y + b_ref[0]
    y = y * jax.nn.sigmoid(y)
    out_ref[...] = wpad_ref[...] * y


def _final_kernel(x_ref, wsh_ref, bsh_ref, g0_ref, g1_ref, out_ref):
    y = jnp.dot(x_ref[...], wsh_ref[...], preferred_element_type=jnp.float32)
    y = y + bsh_ref[0]
    y = y * jax.nn.sigmoid(y)
    out_ref[...] = y + g0_ref[...] + g1_ref[...]


def _dispatch_plan(topk_idx, topk_w, blk):
    n = topk_idx.shape[0]
    m = n * TOPK
    p_total = m + N_EXP * blk
    flat_idx = topk_idx.reshape(m).astype(jnp.int32)
    flat_w = topk_w.reshape(m)
    tok = jax.lax.iota(jnp.int32, m) // TOPK

    onehot = (flat_idx[:, None] == jnp.arange(N_EXP, dtype=jnp.int32)[None, :])
    csum = jnp.cumsum(onehot.astype(jnp.int32), axis=0)
    rank = jnp.take_along_axis(csum, flat_idx[:, None], axis=1)[:, 0] - 1
    counts = csum[-1]
    padded_counts = ((counts + blk - 1) // blk) * blk
    pstart = jnp.concatenate([jnp.zeros((1,), jnp.int32),
                              jnp.cumsum(padded_counts)[:-1]])
    dpos = pstart[flat_idx] + rank

    gather_idx = jnp.zeros((p_total,), jnp.int32).at[dpos].set(tok)
    w_pad = jnp.zeros((p_total,), jnp.float32).at[dpos].set(flat_w)
    bounds = jnp.cumsum(padded_counts)
    blk_starts = jax.lax.iota(jnp.int32, p_total // blk) * blk
    block_expert = jnp.minimum(
        jnp.searchsorted(bounds, blk_starts, side='right').astype(jnp.int32),
        N_EXP - 1)
    return gather_idx, w_pad, block_expert, dpos


def _moe_stream(x, topk_w, topk_idx, W, b, Wsh, bsh, blk):
    n, din = x.shape
    dout = W.shape[-1]
    gather_idx, w_pad, block_expert, dpos = _dispatch_plan(topk_idx, topk_w, blk)
    p_total = gather_idx.shape[0]

    xs = jnp.take(x, gather_idx, axis=0)

    grid_spec = pltpu.PrefetchScalarGridSpec(
        num_scalar_prefetch=1,
        grid=(p_total // blk,),
        in_specs=[
            pl.BlockSpec((blk, din), lambda i, be: (i, 0)),
            pl.BlockSpec((1, din, dout), lambda i, be: (be[i], 0, 0)),
            pl.BlockSpec((1, 1, dout), lambda i, be: (be[i], 0, 0)),
            pl.BlockSpec((blk, 1), lambda i, be: (i, 0)),
        ],
        out_specs=pl.BlockSpec((blk, dout), lambda i, be: (i, 0)),
    )
    yw = pl.pallas_call(
        _grouped_matmul_kernel,
        grid_spec=grid_spec,
        out_shape=jax.ShapeDtypeStruct((p_total, dout), jnp.float32),
    )(block_expert, xs, W, b[:, None, :], w_pad[:, None])

    g0 = jnp.take(yw, dpos[0::2], axis=0)
    g1 = jnp.take(yw, dpos[1::2], axis=0)

    blk2 = 512
    return pl.pallas_call(
        _final_kernel,
        grid=(n // blk2,),
        in_specs=[
            pl.BlockSpec((blk2, din), lambda i: (i, 0)),
            pl.BlockSpec((din, dout), lambda i: (0, 0)),
            pl.BlockSpec((1, dout), lambda i: (0, 0)),
            pl.BlockSpec((blk2, dout), lambda i: (i, 0)),
            pl.BlockSpec((blk2, dout), lambda i: (i, 0)),
        ],
        out_specs=pl.BlockSpec((blk2, dout), lambda i: (i, 0)),
        out_shape=jax.ShapeDtypeStruct((n, dout), jnp.float32),
    )(x, Wsh, bsh[None, :], g0, g1)


@jax.jit
def kernel(node_m1_input, node_m2_input, edge_input, angle_input,
           node_router_weights, node_router_indices,
           edge_router_weights, edge_router_indices,
           angle_router_weights, angle_router_indices,
           n2e_index, n2a_index,
           node_self_W, node_self_b, node_self_Wsh, node_self_bsh,
           node_sym_W, node_sym_b, node_sym_Wsh, node_sym_bsh,
           edge_W, edge_b, edge_Wsh, edge_bsh,
           angle_W, angle_b, angle_Wsh, angle_bsh):
    edge_idx = edge_router_indices[n2e_index]
    angle_idx = angle_router_indices[n2a_index]
    edge_w = edge_router_weights[n2e_index]
    angle_w = angle_router_weights[n2a_index]

    node_self_out = _moe_stream(node_m1_input, node_router_weights,
                                node_router_indices, node_self_W, node_self_b,
                                node_self_Wsh, node_self_bsh, 256)
    node_sym_out = _moe_stream(node_m2_input, node_router_weights,
                               node_router_indices, node_sym_W, node_sym_b,
                               node_sym_Wsh, node_sym_bsh, 256)
    edge_out = _moe_stream(edge_input, edge_w, edge_idx, edge_W, edge_b,
                           edge_Wsh, edge_bsh, 256)
    angle_out = _moe_stream(angle_input, angle_w, angle_idx, angle_W, angle_b,
                            angle_Wsh, angle_bsh, 256)
    return node_self_out, node_sym_out, edge_out, angle_out


# speedup vs baseline: 2.3019x; 2.3019x over previous
import functools

import jax
import jax.numpy as jnp
from jax import lax
from jax.experimental import pallas as pl
from jax.experimental.pallas import tpu as pltpu
from jax.experimental.pallas import tpu_sc as plsc

N_EXP = 8
TOPK = 2
NC = 2
NS = 16
NW = NC * NS
CHUNK = 64


def _make_dispatch_kernel(n, din, p_total):
    tpw = n // NW
    n_chunks = tpw // CHUNK
    mesh = plsc.VectorSubcoreMesh(core_axis_name="c", subcore_axis_name="s",
                                  num_cores=NC, num_subcores=NS)

    @functools.partial(
        pl.kernel,
        out_type=jax.ShapeDtypeStruct((p_total, din), jnp.float32),
        mesh=mesh,
        scratch_types=[
            pltpu.VMEM((CHUNK, din), jnp.float32),
            pltpu.VMEM((CHUNK,), jnp.int32),
            pltpu.VMEM((CHUNK,), jnp.int32),
            pltpu.SemaphoreType.DMA,
        ],
    )
    def dispatch(x_hbm, dpos_e_hbm, dpos_o_hbm, xs_hbm, rows_v, idxe_v,
                 idxo_v, sem):
        wid = lax.axis_index("s") * NC + lax.axis_index("c")

        def body(c, _):
            base = wid * tpw + c * CHUNK
            pltpu.sync_copy(x_hbm.at[pl.ds(base, CHUNK)], rows_v)
            pltpu.sync_copy(dpos_e_hbm.at[pl.ds(base, CHUNK)], idxe_v)
            pltpu.sync_copy(dpos_o_hbm.at[pl.ds(base, CHUNK)], idxo_v)
            c1 = pltpu.async_copy(rows_v, xs_hbm.at[idxe_v], sem)
            c2 = pltpu.async_copy(rows_v, xs_hbm.at[idxo_v], sem)
            c1.wait()
            c2.wait()
            return ()

        lax.fori_loop(0, n_chunks, body, ())

    return dispatch


def _make_combine_kernel(n, dout, p_total):
    tpw = n // NW
    n_chunks = tpw // CHUNK
    mesh = plsc.VectorSubcoreMesh(core_axis_name="c", subcore_axis_name="s",
                                  num_cores=NC, num_subcores=NS)

    @functools.partial(
        pl.kernel,
        out_type=(jax.ShapeDtypeStruct((n, dout), jnp.float32),
                  jax.ShapeDtypeStruct((n, dout), jnp.float32)),
        mesh=mesh,
        scratch_types=[
            pltpu.VMEM((CHUNK, dout), jnp.float32),
            pltpu.VMEM((CHUNK, dout), jnp.float32),
            pltpu.VMEM((CHUNK,), jnp.int32),
            pltpu.VMEM((CHUNK,), jnp.int32),
            pltpu.SemaphoreType.DMA,
        ],
    )
    def combine(yw_hbm, dpos_e_hbm, dpos_o_hbm, g0_hbm, g1_hbm, rows0_v,
                rows1_v, idxe_v, idxo_v, sem):
        wid = lax.axis_index("s") * NC + lax.axis_index("c")

        def body(c, _):
            base = wid * tpw + c * CHUNK
            pltpu.sync_copy(dpos_e_hbm.at[pl.ds(base, CHUNK)], idxe_v)
            pltpu.sync_copy(dpos_o_hbm.at[pl.ds(base, CHUNK)], idxo_v)
            c1 = pltpu.async_copy(yw_hbm.at[idxe_v], rows0_v, sem)
            c2 = pltpu.async_copy(yw_hbm.at[idxo_v], rows1_v, sem)
            c1.wait()
            c2.wait()
            pltpu.sync_copy(rows0_v, g0_hbm.at[pl.ds(base, CHUNK)])
            pltpu.sync_copy(rows1_v, g1_hbm.at[pl.ds(base, CHUNK)])
            return ()

        lax.fori_loop(0, n_chunks, body, ())

    return combine


def _grouped_matmul_kernel(be_ref, xs_ref, w_ref, b_ref, out_ref):
    y = jnp.dot(xs_ref[...], w_ref[0], preferred_element_type=jnp.float32)
    y = y + b_ref[0]
    out_ref[...] = y * jax.nn.sigmoid(y)


def _final_kernel(x_ref, wsh_ref, bsh_ref, g0_ref, g1_ref, w0_ref, w1_ref,
                  out_ref):
    y = jnp.dot(x_ref[...], wsh_ref[...], preferred_element_type=jnp.float32)
    y = y + bsh_ref[0]
    y = y * jax.nn.sigmoid(y)
    out_ref[...] = y + w0_ref[...] * g0_ref[...] + w1_ref[...] * g1_ref[...]


def _dispatch_plan(topk_idx, blk):
    n = topk_idx.shape[0]
    m = n * TOPK
    seg = 256
    p_total = m + N_EXP * blk
    flat_idx = topk_idx.reshape(m).astype(jnp.int32)

    oh = (flat_idx[:, None] == jnp.arange(N_EXP, dtype=jnp.int32)[None, :])
    oh = oh.astype(jnp.float32)
    oh3 = oh.reshape(m // seg, seg, N_EXP)
    tri = jnp.tril(jnp.ones((seg, seg), jnp.float32))
    within = jnp.einsum('ts,bso->bto', tri, oh3,
                        preferred_element_type=jnp.float32)
    bsum = oh3.sum(axis=1)
    excl = jnp.cumsum(bsum, axis=0) - bsum
    incl = (within + excl[:, None, :]).reshape(m, N_EXP)
    rank = (incl * oh).sum(axis=1) - 1.0

    counts = bsum.sum(axis=0)
    padded_counts = jnp.ceil(counts / blk) * blk
    pstart = jnp.cumsum(padded_counts) - padded_counts
    dpos = ((pstart[None, :] * oh).sum(axis=1) + rank).astype(jnp.int32)

    bounds = jnp.cumsum(padded_counts).astype(jnp.int32)
    blk_starts = lax.iota(jnp.int32, p_total // blk) * blk
    block_expert = jnp.minimum(
        jnp.searchsorted(bounds, blk_starts, side='right').astype(jnp.int32),
        N_EXP - 1)
    return dpos, block_expert


def _moe_stream(x, topk_w, topk_idx, W, b, Wsh, bsh, blk):
    n, din = x.shape
    dout0 = W.shape[-1]
    dout = ((dout0 + 127) // 128) * 128
    if dout != dout0:
        pad = dout - dout0
        W = jnp.pad(W, ((0, 0), (0, 0), (0, pad)))
        b = jnp.pad(b, ((0, 0), (0, pad)))
        Wsh = jnp.pad(Wsh, ((0, 0), (0, pad)))
        bsh = jnp.pad(bsh, ((0, pad),))
    dpos, block_expert = _dispatch_plan(topk_idx, blk)
    p_total = n * TOPK + N_EXP * blk
    dpos_e = dpos[0::2]
    dpos_o = dpos[1::2]

    xs = _make_dispatch_kernel(n, din, p_total)(x, dpos_e, dpos_o)

    grid_spec = pltpu.PrefetchScalarGridSpec(
        num_scalar_prefetch=1,
        grid=(p_total // blk,),
        in_specs=[
            pl.BlockSpec((blk, din), lambda i, be: (i, 0)),
            pl.BlockSpec((1, din, dout), lambda i, be: (be[i], 0, 0)),
            pl.BlockSpec((1, 1, dout), lambda i, be: (be[i], 0, 0)),
        ],
        out_specs=pl.BlockSpec((blk, dout), lambda i, be: (i, 0)),
    )
    yw = pl.pallas_call(
        _grouped_matmul_kernel,
        grid_spec=grid_spec,
        out_shape=jax.ShapeDtypeStruct((p_total, dout), jnp.float32),
    )(block_expert, xs, W, b[:, None, :])

    g0, g1 = _make_combine_kernel(n, dout, p_total)(yw, dpos_e, dpos_o)

    blk2 = 512
    out = pl.pallas_call(
        _final_kernel,
        grid=(n // blk2,),
        in_specs=[
            pl.BlockSpec((blk2, din), lambda i: (i, 0)),
            pl.BlockSpec((din, dout), lambda i: (0, 0)),
            pl.BlockSpec((1, dout), lambda i: (0, 0)),
            pl.BlockSpec((blk2, dout), lambda i: (i, 0)),
            pl.BlockSpec((blk2, dout), lambda i: (i, 0)),
            pl.BlockSpec((blk2, 1), lambda i: (i, 0)),
            pl.BlockSpec((blk2, 1), lambda i: (i, 0)),
        ],
        out_specs=pl.BlockSpec((blk2, dout), lambda i: (i, 0)),
        out_shape=jax.ShapeDtypeStruct((n, dout), jnp.float32),
    )(x, Wsh, bsh[None, :], g0, g1, topk_w[:, 0:1], topk_w[:, 1:2])
    return out[:, :dout0] if dout != dout0 else out


@jax.jit
def kernel(node_m1_input, node_m2_input, edge_input, angle_input,
           node_router_weights, node_router_indices,
           edge_router_weights, edge_router_indices,
           angle_router_weights, angle_router_indices,
           n2e_index, n2a_index,
           node_self_W, node_self_b, node_self_Wsh, node_self_bsh,
           node_sym_W, node_sym_b, node_sym_Wsh, node_sym_bsh,
           edge_W, edge_b, edge_Wsh, edge_bsh,
           angle_W, angle_b, angle_Wsh, angle_bsh):
    edge_idx = edge_router_indices[n2e_index]
    angle_idx = angle_router_indices[n2a_index]
    edge_w = edge_router_weights[n2e_index]
    angle_w = angle_router_weights[n2a_index]

    node_self_out = _moe_stream(node_m1_input, node_router_weights,
                                node_router_indices, node_self_W, node_self_b,
                                node_self_Wsh, node_self_bsh, 256)
    node_sym_out = _moe_stream(node_m2_input, node_router_weights,
                               node_router_indices, node_sym_W, node_sym_b,
                               node_sym_Wsh, node_sym_bsh, 256)
    edge_out = _moe_stream(edge_input, edge_w, edge_idx, edge_W, edge_b,
                           edge_Wsh, edge_bsh, 256)
    angle_out = _moe_stream(angle_input, angle_w, angle_idx, angle_W, angle_b,
                            angle_Wsh, angle_bsh, 256)
    return node_self_out, node_sym_out, edge_out, angle_out
